# baseline (speedup 1.0000x reference)
import functools

import jax
import jax.numpy as jnp
from jax import lax
from jax.experimental import pallas as pl
from jax.experimental.pallas import tpu as pltpu
from jax.experimental.pallas import tpu_sc as plsc

_N = 10000
_E = 320000
_ND = 128
_ED = 16
_H = 128

_info = plsc.get_sparse_core_info()
_NC = _info.num_cores
_NS = _info.num_subcores
_NW = _NC * _NS
_EPW = _E // _NW
_K = 80
_NB = _EPW // _K
_NPAD = 10240
_ZR = _NPAD // _NS

_mesh = plsc.VectorSubcoreMesh(core_axis_name="c", subcore_axis_name="s")


def _pre_body(x_ref, wa_ref, wb_ref, p_ref, q_ref):
    x = x_ref[...]
    p_ref[...] = jnp.dot(x, wa_ref[...],
                         preferred_element_type=jnp.float32).astype(jnp.bfloat16)
    q_ref[...] = jnp.dot(x, wb_ref[...],
                         preferred_element_type=jnp.float32).astype(jnp.bfloat16)


def _precompute(x, wa, wb):
    return pl.pallas_call(
        _pre_body,
        out_shape=(
            jax.ShapeDtypeStruct((_N, _ND), jnp.bfloat16),
            jax.ShapeDtypeStruct((_N, _ND), jnp.bfloat16),
        ),
    )(x, wa, wb)


@functools.partial(
    pl.kernel,
    out_type=(
        jax.ShapeDtypeStruct((_E, _ND), jnp.bfloat16),
        jax.ShapeDtypeStruct((_E, _ND), jnp.bfloat16),
    ),
    mesh=_mesh,
    scratch_types=[
        pltpu.VMEM((_K,), jnp.int32),
        pltpu.VMEM((_K,), jnp.int32),
        pltpu.VMEM((_K, _ND), jnp.bfloat16),
        pltpu.VMEM((_K, _ND), jnp.bfloat16),
        pltpu.SemaphoreType.DMA,
        pltpu.SemaphoreType.DMA,
    ],
    compiler_params=pltpu.CompilerParams(use_tc_tiling_on_sc=False),
)
def _sc_gather(p_hbm, q_hbm, src_hbm, dst_hbm, g1_hbm, g2_hbm,
               ia, ib, ra, rb, sem1, sem2):
    wid = lax.axis_index("s") * _NC + lax.axis_index("c")
    base = pl.multiple_of(wid * _EPW, 8)

    def body(b, carry):
        off = pl.multiple_of(base + b * _K, 8)
        pltpu.sync_copy(src_hbm.at[pl.ds(off, _K)], ia)
        pltpu.sync_copy(dst_hbm.at[pl.ds(off, _K)], ib)
        c1 = pltpu.async_copy(p_hbm.at[ia], ra, sem1)
        c2 = pltpu.async_copy(q_hbm.at[ib], rb, sem2)
        c1.wait()
        c2.wait()
        pltpu.sync_copy(ra, g1_hbm.at[pl.ds(off, _K)])
        pltpu.sync_copy(rb, g2_hbm.at[pl.ds(off, _K)])
        return carry

    lax.fori_loop(0, _NB, body, 0)


def _edge_body(g1_ref, g2_ref, ea_ref, wc_ref, be1_ref, we2_ref, be2_ref,
               ge_ref, beln_ref, out_ref):
    ea = ea_ref[...]
    s = (g1_ref[...].astype(jnp.float32) + g2_ref[...].astype(jnp.float32)
         + be1_ref[...]
         + jnp.dot(ea, wc_ref[...], preferred_element_type=jnp.float32))
    h = jnp.maximum(s, 0.0)
    y = jnp.dot(h, we2_ref[...], preferred_element_type=jnp.float32) + be2_ref[...]
    mu = jnp.mean(y, axis=-1, keepdims=True)
    var = jnp.mean((y - mu) ** 2, axis=-1, keepdims=True)
    yn = (y - mu) * lax.rsqrt(var + 1e-5) * ge_ref[...] + beln_ref[...]
    out_ref[...] = ea + yn


def _edge_mlp(g1, g2, edge_attr, wc, be1, we2, be2, ge_ln, be_ln):
    blk = 4000
    grid = _E // blk
    full = lambda s: pl.BlockSpec(s, lambda i: (0, 0))
    return pl.pallas_call(
        _edge_body,
        grid=(grid,),
        in_specs=[
            pl.BlockSpec((blk, _ND), lambda i: (i, 0)),
            pl.BlockSpec((blk, _ND), lambda i: (i, 0)),
            pl.BlockSpec((blk, _ED), lambda i: (i, 0)),
            full((_ED, _H)),
            full((1, _H)),
            full((_H, _ED)),
            full((1, _ED)),
            full((1, _ED)),
            full((1, _ED)),
        ],
        out_specs=pl.BlockSpec((blk, _ED), lambda i: (i, 0)),
        out_shape=jax.ShapeDtypeStruct((_E, _ED), jnp.float32),
    )(g1, g2, edge_attr, wc, be1, we2, be2, ge_ln, be_ln)


@functools.partial(
    pl.kernel,
    out_type=jax.ShapeDtypeStruct((_NC, _NPAD, _ED), jnp.float32),
    mesh=_mesh,
    scratch_types=[
        pltpu.VMEM((_K,), jnp.int32),
        pltpu.VMEM((_K, _ED), jnp.float32),
        pltpu.VMEM_SHARED((_NPAD, _ED), jnp.float32),
    ],
    compiler_params=pltpu.CompilerParams(use_tc_tiling_on_sc=False),
)
def _sc_scatter(zin_hbm, dst_hbm, ea_hbm, out_hbm, idx, rows, acc):
    cid = lax.axis_index("c")
    sid = lax.axis_index("s")
    wid = sid * _NC + cid
    base = pl.multiple_of(wid * _EPW, 8)
    stripe = sid * _ZR

    pltpu.sync_copy(zin_hbm.at[pl.ds(stripe, _ZR)], acc.at[pl.ds(stripe, _ZR)])
    plsc.subcore_barrier()

    def body(b, carry):
        off = pl.multiple_of(base + b * _K, 8)
        pltpu.sync_copy(dst_hbm.at[pl.ds(off, _K)], idx)
        pltpu.sync_copy(ea_hbm.at[pl.ds(off, _K)], rows)
        pltpu.sync_copy(rows, acc.at[idx], add=True)
        return carry

    lax.fori_loop(0, _NB, body, 0)
    plsc.subcore_barrier()

    pltpu.sync_copy(acc.at[pl.ds(stripe, _ZR)], out_hbm.at[cid, pl.ds(stripe, _ZR)])


def _node_body(x_ref, pa_ref, wn1a_ref, wn1b_ref, bn1_ref, wn2_ref, bn2_ref,
               gn_ref, bnln_ref, out_ref):
    x = x_ref[...]
    aggr = (pa_ref[0] + pa_ref[1])[:_N]
    s = (jnp.dot(x, wn1a_ref[...], preferred_element_type=jnp.float32)
         + jnp.dot(aggr, wn1b_ref[...], preferred_element_type=jnp.float32)
         + bn1_ref[...])
    h = jnp.maximum(s, 0.0)
    y = jnp.dot(h, wn2_ref[...], preferred_element_type=jnp.float32) + bn2_ref[...]
    mu = jnp.mean(y, axis=-1, keepdims=True)
    var = jnp.mean((y - mu) ** 2, axis=-1, keepdims=True)
    yn = (y - mu) * lax.rsqrt(var + 1e-5) * gn_ref[...] + bnln_ref[...]
    out_ref[...] = x + yn


def _node_mlp(x, partials, wn1a, wn1b, bn1, wn2, bn2, gn_ln, bn_ln):
    return pl.pallas_call(
        _node_body,
        out_shape=jax.ShapeDtypeStruct((_N, _ND), jnp.float32),
    )(x, partials, wn1a, wn1b, bn1, wn2, bn2, gn_ln, bn_ln)


def kernel(x, edge_attr, edge_index, We1, be1, We2, be2, ge_ln, be_ln,
           Wn1, bn1, Wn2, bn2, gn_ln, bn_ln):
    src = edge_index[0]
    dst = edge_index[1]
    wa = We1[:_ND]
    wb = We1[_ND:2 * _ND]
    wc = We1[2 * _ND:]

    p, q = _precompute(x, wa, wb)
    g1, g2 = _sc_gather(p, q, src, dst)
    edge_attr_upd = _edge_mlp(
        g1, g2, edge_attr, wc,
        be1.reshape(1, _H), We2, be2.reshape(1, _ED),
        ge_ln.reshape(1, _ED), be_ln.reshape(1, _ED))
    zin = jnp.zeros((_NPAD, _ED), jnp.float32)
    partials = _sc_scatter(zin, dst, edge_attr_upd)
    x_upd = _node_mlp(
        x, partials, Wn1[:_ND], Wn1[_ND:], bn1.reshape(1, _H),
        Wn2, bn2.reshape(1, _ND), gn_ln.reshape(1, _ND), bn_ln.reshape(1, _ND))
    return (x_upd, edge_attr_upd)


# speedup vs baseline: 1.7132x; 1.7132x over previous
import functools

import jax
import jax.numpy as jnp
from jax import lax
from jax.experimental import pallas as pl
from jax.experimental.pallas import tpu as pltpu
from jax.experimental.pallas import tpu_sc as plsc

_N = 10000
_E = 320000
_ND = 128
_ED = 16
_H = 128

_info = plsc.get_sparse_core_info()
_NC = _info.num_cores
_NS = _info.num_subcores
_NW = _NC * _NS
_EPW = _E // _NW
_K = 80
_NB = _EPW // _K
_NPAD = 10240
_ZR = _NPAD // _NS

_mesh = plsc.VectorSubcoreMesh(core_axis_name="c", subcore_axis_name="s")


def _pre_body(x_ref, wa_ref, wb_ref, p_ref, q_ref):
    x = x_ref[...]
    p_ref[...] = jnp.dot(x, wa_ref[...], preferred_element_type=jnp.float32)
    q_ref[...] = jnp.dot(x, wb_ref[...], preferred_element_type=jnp.float32)


def _precompute(x, wa, wb):
    return pl.pallas_call(
        _pre_body,
        out_shape=(
            jax.ShapeDtypeStruct((_N, _ND), jnp.float32),
            jax.ShapeDtypeStruct((_N, _ND), jnp.float32),
        ),
    )(x, wa, wb)


@functools.partial(
    pl.kernel,
    out_type=(
        jax.ShapeDtypeStruct((_E, _ND), jnp.float32),
        jax.ShapeDtypeStruct((_E, _ND), jnp.float32),
    ),
    mesh=_mesh,
    scratch_types=[
        pltpu.VMEM((_K,), jnp.int32),
        pltpu.VMEM((_K,), jnp.int32),
        pltpu.VMEM((_K, _ND), jnp.float32),
        pltpu.VMEM((_K, _ND), jnp.float32),
        pltpu.SemaphoreType.DMA,
        pltpu.SemaphoreType.DMA,
        pltpu.SemaphoreType.DMA,
        pltpu.SemaphoreType.DMA,
    ],
)
def _sc_gather(p_hbm, q_hbm, src_hbm, dst_hbm, g1_hbm, g2_hbm,
               ia, ib, ra, rb, sem1, sem2, sem3, sem4):
    wid = lax.axis_index("s") * _NC + lax.axis_index("c")
    base = pl.multiple_of(wid * _EPW, 8)

    def body(b, carry):
        off = pl.multiple_of(base + b * _K, 8)
        pltpu.sync_copy(src_hbm.at[pl.ds(off, _K)], ia)
        pltpu.sync_copy(dst_hbm.at[pl.ds(off, _K)], ib)

        @pl.when(b > 0)
        def _():
            prev = pl.multiple_of(base + (b - 1) * _K, 8)
            pltpu.make_async_copy(ra, g1_hbm.at[pl.ds(prev, _K)], sem3).wait()
            pltpu.make_async_copy(rb, g2_hbm.at[pl.ds(prev, _K)], sem4).wait()

        c1 = pltpu.async_copy(p_hbm.at[ia], ra, sem1)
        c2 = pltpu.async_copy(q_hbm.at[ib], rb, sem2)
        c1.wait()
        c2.wait()
        pltpu.async_copy(ra, g1_hbm.at[pl.ds(off, _K)], sem3)
        pltpu.async_copy(rb, g2_hbm.at[pl.ds(off, _K)], sem4)
        return carry

    lax.fori_loop(0, _NB, body, 0)
    last = pl.multiple_of(base + (_NB - 1) * _K, 8)
    pltpu.make_async_copy(ra, g1_hbm.at[pl.ds(last, _K)], sem3).wait()
    pltpu.make_async_copy(rb, g2_hbm.at[pl.ds(last, _K)], sem4).wait()


def _edge_body(g1_ref, g2_ref, ea_ref, wc_ref, be1_ref, we2_ref, be2_ref,
               ge_ref, beln_ref, out_ref):
    ea = ea_ref[...]
    s = (g1_ref[...] + g2_ref[...] + be1_ref[...]
         + jnp.dot(ea, wc_ref[...], preferred_element_type=jnp.float32))
    h = jnp.maximum(s, 0.0)
    y = jnp.dot(h, we2_ref[...], preferred_element_type=jnp.float32) + be2_ref[...]
    mu = jnp.mean(y, axis=-1, keepdims=True)
    var = jnp.mean((y - mu) ** 2, axis=-1, keepdims=True)
    yn = (y - mu) * lax.rsqrt(var + 1e-5) * ge_ref[...] + beln_ref[...]
    out_ref[...] = ea + yn


def _edge_mlp(g1, g2, edge_attr, wc, be1, we2, be2, ge_ln, be_ln):
    blk = 4000
    grid = _E // blk
    full = lambda s: pl.BlockSpec(s, lambda i: (0, 0))
    return pl.pallas_call(
        _edge_body,
        grid=(grid,),
        in_specs=[
            pl.BlockSpec((blk, _ND), lambda i: (i, 0)),
            pl.BlockSpec((blk, _ND), lambda i: (i, 0)),
            pl.BlockSpec((blk, _ED), lambda i: (i, 0)),
            full((_ED, _H)),
            full((1, _H)),
            full((_H, _ED)),
            full((1, _ED)),
            full((1, _ED)),
            full((1, _ED)),
        ],
        out_specs=pl.BlockSpec((blk, _ED), lambda i: (i, 0)),
        out_shape=jax.ShapeDtypeStruct((_E, _ED), jnp.float32),
    )(g1, g2, edge_attr, wc, be1, we2, be2, ge_ln, be_ln)


@functools.partial(
    pl.kernel,
    out_type=jax.ShapeDtypeStruct((_NC, _NPAD, _ED), jnp.float32),
    mesh=_mesh,
    scratch_types=[
        pltpu.VMEM((_K,), jnp.int32),
        pltpu.VMEM((_K, _ED), jnp.float32),
        pltpu.VMEM_SHARED((_NPAD, _ED), jnp.float32),
    ],
    compiler_params=pltpu.CompilerParams(use_tc_tiling_on_sc=False),
)
def _sc_scatter(zin_hbm, dst_hbm, ea_hbm, out_hbm, idx, rows, acc):
    cid = lax.axis_index("c")
    sid = lax.axis_index("s")
    wid = sid * _NC + cid
    base = pl.multiple_of(wid * _EPW, 8)
    stripe = sid * _ZR

    pltpu.sync_copy(zin_hbm.at[pl.ds(stripe, _ZR)], acc.at[pl.ds(stripe, _ZR)])
    plsc.subcore_barrier()

    def body(b, carry):
        off = pl.multiple_of(base + b * _K, 8)
        pltpu.sync_copy(dst_hbm.at[pl.ds(off, _K)], idx)
        pltpu.sync_copy(ea_hbm.at[pl.ds(off, _K)], rows)
        pltpu.sync_copy(rows, acc.at[idx], add=True)
        return carry

    lax.fori_loop(0, _NB, body, 0)
    plsc.subcore_barrier()

    pltpu.sync_copy(acc.at[pl.ds(stripe, _ZR)], out_hbm.at[cid, pl.ds(stripe, _ZR)])


def _node_body(x_ref, pa_ref, wn1a_ref, wn1b_ref, bn1_ref, wn2_ref, bn2_ref,
               gn_ref, bnln_ref, out_ref):
    x = x_ref[...]
    aggr = (pa_ref[0] + pa_ref[1])[:_N]
    s = (jnp.dot(x, wn1a_ref[...], preferred_element_type=jnp.float32)
         + jnp.dot(aggr, wn1b_ref[...], preferred_element_type=jnp.float32)
         + bn1_ref[...])
    h = jnp.maximum(s, 0.0)
    y = jnp.dot(h, wn2_ref[...], preferred_element_type=jnp.float32) + bn2_ref[...]
    mu = jnp.mean(y, axis=-1, keepdims=True)
    var = jnp.mean((y - mu) ** 2, axis=-1, keepdims=True)
    yn = (y - mu) * lax.rsqrt(var + 1e-5) * gn_ref[...] + bnln_ref[...]
    out_ref[...] = x + yn


def _node_mlp(x, partials, wn1a, wn1b, bn1, wn2, bn2, gn_ln, bn_ln):
    return pl.pallas_call(
        _node_body,
        out_shape=jax.ShapeDtypeStruct((_N, _ND), jnp.float32),
    )(x, partials, wn1a, wn1b, bn1, wn2, bn2, gn_ln, bn_ln)


def kernel(x, edge_attr, edge_index, We1, be1, We2, be2, ge_ln, be_ln,
           Wn1, bn1, Wn2, bn2, gn_ln, bn_ln):
    src = edge_index[0]
    dst = edge_index[1]
    wa = We1[:_ND]
    wb = We1[_ND:2 * _ND]
    wc = We1[2 * _ND:]

    p, q = _precompute(x, wa, wb)
    g1, g2 = _sc_gather(p, q, src, dst)
    edge_attr_upd = _edge_mlp(
        g1, g2, edge_attr, wc,
        be1.reshape(1, _H), We2, be2.reshape(1, _ED),
        ge_ln.reshape(1, _ED), be_ln.reshape(1, _ED))
    zin = jnp.zeros((_NPAD, _ED), jnp.float32)
    partials = _sc_scatter(zin, dst, edge_attr_upd)
    x_upd = _node_mlp(
        x, partials, Wn1[:_ND], Wn1[_ND:], bn1.reshape(1, _H),
        Wn2, bn2.reshape(1, _ND), gn_ln.reshape(1, _ND), bn_ln.reshape(1, _ND))
    return (x_upd, edge_attr_upd)
